# baseline (speedup 1.0000x reference)
import functools

import jax
import jax.numpy as jnp
from jax import lax
from jax.experimental import pallas as pl
from jax.experimental.pallas import tpu as pltpu
from jax.experimental.pallas import tpu_sc as plsc

NC, NS, L = 2, 16, 16
NW = NC * NS
HIDDEN = 128
ROWS_PER_STEP = 4096


def _make_sc_gather(n_tokens):
    b_per_w = n_tokens // NW
    mesh = plsc.VectorSubcoreMesh(
        core_axis_name="c", subcore_axis_name="s", num_cores=NC, num_subcores=NS
    )

    @functools.partial(
        pl.kernel,
        mesh=mesh,
        compiler_params=pltpu.CompilerParams(needs_layout_passes=False, skip_device_barrier=True, disable_bounds_checks=True, disable_semaphore_checks=True),
        out_type=jax.ShapeDtypeStruct((n_tokens, HIDDEN), jnp.float32),
        scratch_types=[
            pltpu.VMEM((b_per_w,), jnp.int32),
            pltpu.VMEM((b_per_w, HIDDEN), jnp.float32),
            pltpu.SemaphoreType.DMA,
            pltpu.SemaphoreType.DMA,
        ],
    )
    def sc_gather(ids_hbm, token_hbm, out_hbm, idx_v, rows_v, sem0, sem1):
        wid = lax.axis_index("s") * NC + lax.axis_index("c")
        base = wid * b_per_w
        nchunks = b_per_w // 128
        id_copies = [
            pltpu.async_copy(ids_hbm.at[pl.ds(base + j * 128, 128)],
                             idx_v.at[pl.ds(j * 128, 128)], sem0)
            for j in range(nchunks)]
        gathers = []
        for j in range(nchunks):
            id_copies[j].wait()
            gathers.append(pltpu.async_copy(
                token_hbm.at[idx_v.at[pl.ds(j * 128, 128)]],
                rows_v.at[pl.ds(j * 128, 128)], sem1))
        outs = []
        for j, g in enumerate(gathers):
            g.wait()
            outs.append(pltpu.async_copy(
                rows_v.at[pl.ds(j * 128, 128)],
                out_hbm.at[pl.ds(base + j * 128, 128)], sem0))
        for o in outs:
            o.wait()

    return sc_gather


def _tc_ln_body(x_ref, pos_ref, ttf_ref, type_ref, g_ref, b_ref, o_ref):
    x = x_ref[...]
    t0 = type_ref[0:1, :]
    t1 = type_ref[1:2, :]
    n_rep = x.shape[0] // pos_ref.shape[0]
    pos = pos_ref[...]
    if n_rep > 1:
        pos = jnp.concatenate([pos] * n_rep, axis=0)
    e = x + pos + t0 + ttf_ref[...] * (t1 - t0)
    mean = jnp.mean(e, axis=-1, keepdims=True)
    c = e - mean
    var = jnp.mean(c * c, axis=-1, keepdims=True)
    o_ref[...] = c * lax.rsqrt(var + 1e-12) * g_ref[...] + b_ref[...]


def _tc_ln(gathered, pos_table, ttf, type_table, gamma, beta, seq_len):
    n = gathered.shape[0]
    r = ROWS_PER_STEP
    grid = n // r
    pos_blocks = seq_len // r if seq_len >= r else 1
    return pl.pallas_call(
        _tc_ln_body,
        grid=(grid,),
        in_specs=[
            pl.BlockSpec((r, HIDDEN), lambda g: (g, 0)),
            pl.BlockSpec((min(r, seq_len), HIDDEN), lambda g: (g % pos_blocks, 0)),
            pl.BlockSpec((r, 1), lambda g: (g, 0)),
            pl.BlockSpec((2, HIDDEN), lambda g: (0, 0)),
            pl.BlockSpec((1, HIDDEN), lambda g: (0, 0)),
            pl.BlockSpec((1, HIDDEN), lambda g: (0, 0)),
        ],
        out_specs=pl.BlockSpec((r, HIDDEN), lambda g: (g, 0)),
        out_shape=jax.ShapeDtypeStruct((n, HIDDEN), jnp.float32),
    )(gathered, pos_table, ttf, type_table, gamma, beta)


def kernel(input_ids, token_type_ids, token_table, pos_table, type_table,
           ln_gamma, ln_beta):
    b, s = input_ids.shape
    n = b * s
    ids = input_ids.reshape(n).astype(jnp.int32)
    ttf = token_type_ids.reshape(n, 1).astype(jnp.float32)
    gathered = _make_sc_gather(n)(ids, token_table)
    out = _tc_ln(gathered, pos_table, ttf, type_table,
                 ln_gamma.reshape(1, HIDDEN), ln_beta.reshape(1, HIDDEN), s)
    return out.reshape(b, s, HIDDEN)


# speedup vs baseline: 1.0199x; 1.0199x over previous
import functools

import jax
import jax.numpy as jnp
from jax import lax
from jax.experimental import pallas as pl
from jax.experimental.pallas import tpu as pltpu
from jax.experimental.pallas import tpu_sc as plsc

NC, NS, L = 1, 16, 16
NW = NC * NS
HIDDEN = 128
ROWS_PER_STEP = 4096


def _make_sc_gather(n_tokens):
    b_per_w = n_tokens // NW
    mesh = plsc.VectorSubcoreMesh(
        core_axis_name="c", subcore_axis_name="s", num_cores=NC, num_subcores=NS
    )

    @functools.partial(
        pl.kernel,
        mesh=mesh,
        compiler_params=pltpu.CompilerParams(needs_layout_passes=False, skip_device_barrier=True, disable_bounds_checks=True, disable_semaphore_checks=True),
        out_type=jax.ShapeDtypeStruct((n_tokens, HIDDEN), jnp.float32),
        scratch_types=[
            pltpu.VMEM((b_per_w,), jnp.int32),
            pltpu.VMEM((b_per_w, HIDDEN), jnp.float32),
            pltpu.SemaphoreType.DMA,
            pltpu.SemaphoreType.DMA,
        ],
    )
    def sc_gather(ids_hbm, token_hbm, out_hbm, idx_v, rows_v, sem0, sem1):
        wid = lax.axis_index("s") * NC + lax.axis_index("c")
        base = wid * b_per_w
        nchunks = b_per_w // 128
        id_copies = [
            pltpu.async_copy(ids_hbm.at[pl.ds(base + j * 128, 128)],
                             idx_v.at[pl.ds(j * 128, 128)], sem0)
            for j in range(nchunks)]
        gathers = []
        for j in range(nchunks):
            id_copies[j].wait()
            gathers.append(pltpu.async_copy(
                token_hbm.at[idx_v.at[pl.ds(j * 128, 128)]],
                rows_v.at[pl.ds(j * 128, 128)], sem1))
        outs = []
        for j, g in enumerate(gathers):
            g.wait()
            outs.append(pltpu.async_copy(
                rows_v.at[pl.ds(j * 128, 128)],
                out_hbm.at[pl.ds(base + j * 128, 128)], sem0))
        for o in outs:
            o.wait()

    return sc_gather


def _tc_ln_body(x_ref, pos_ref, ttf_ref, type_ref, g_ref, b_ref, o_ref):
    x = x_ref[...]
    t0 = type_ref[0:1, :]
    t1 = type_ref[1:2, :]
    n_rep = x.shape[0] // pos_ref.shape[0]
    pos = pos_ref[...]
    if n_rep > 1:
        pos = jnp.concatenate([pos] * n_rep, axis=0)
    e = x + pos + t0 + ttf_ref[...].astype(jnp.float32) * (t1 - t0)
    mean = jnp.mean(e, axis=-1, keepdims=True)
    c = e - mean
    var = jnp.mean(c * c, axis=-1, keepdims=True)
    o_ref[...] = c * lax.rsqrt(var + 1e-12) * g_ref[...] + b_ref[...]


def _tc_ln(gathered, pos_table, ttf, type_table, gamma, beta, seq_len):
    n = gathered.shape[0]
    r = ROWS_PER_STEP
    grid = n // r
    pos_blocks = seq_len // r if seq_len >= r else 1
    return pl.pallas_call(
        _tc_ln_body,
        grid=(grid,),
        in_specs=[
            pl.BlockSpec((r, HIDDEN), lambda g: (g, 0)),
            pl.BlockSpec((min(r, seq_len), HIDDEN), lambda g: (g % pos_blocks, 0)),
            pl.BlockSpec((r, 1), lambda g: (g, 0)),
            pl.BlockSpec((2, HIDDEN), lambda g: (0, 0)),
            pl.BlockSpec((1, HIDDEN), lambda g: (0, 0)),
            pl.BlockSpec((1, HIDDEN), lambda g: (0, 0)),
        ],
        out_specs=pl.BlockSpec((r, HIDDEN), lambda g: (g, 0)),
        out_shape=jax.ShapeDtypeStruct((n, HIDDEN), jnp.float32),
    )(gathered, pos_table, ttf, type_table, gamma, beta)


def kernel(input_ids, token_type_ids, token_table, pos_table, type_table,
           ln_gamma, ln_beta):
    b, s = input_ids.shape
    n = b * s
    ids = input_ids.reshape(n).astype(jnp.int32)
    ttf = token_type_ids.reshape(n, 1)
    gathered = _make_sc_gather(n)(ids, token_table)
    out = _tc_ln(gathered, pos_table, ttf, type_table,
                 ln_gamma.reshape(1, HIDDEN), ln_beta.reshape(1, HIDDEN), s)
    return out.reshape(b, s, HIDDEN)


# speedup vs baseline: 1.0222x; 1.0023x over previous
import functools

import jax
import jax.numpy as jnp
from jax import lax
from jax.experimental import pallas as pl
from jax.experimental.pallas import tpu as pltpu
from jax.experimental.pallas import tpu_sc as plsc

NC, NS = 1, 16
NW = NC * NS
HIDDEN = 128
ROWS_PER_STEP = 4096
CHUNK = 128


def _make_sc_gather(n_tokens):
    b_per_w = n_tokens // NW
    mesh = plsc.VectorSubcoreMesh(
        core_axis_name="c", subcore_axis_name="s", num_cores=NC, num_subcores=NS
    )

    @functools.partial(
        pl.kernel,
        mesh=mesh,
        compiler_params=pltpu.CompilerParams(needs_layout_passes=False),
        out_type=jax.ShapeDtypeStruct((n_tokens, HIDDEN), jnp.float32),
        scratch_types=[
            pltpu.VMEM((b_per_w,), jnp.int32),
            pltpu.VMEM((b_per_w, HIDDEN), jnp.float32),
            pltpu.SemaphoreType.DMA,
            pltpu.SemaphoreType.DMA,
        ],
    )
    def sc_gather(ids_hbm, token_hbm, out_hbm, idx_v, rows_v, sem0, sem1):
        wid = lax.axis_index("s") * NC + lax.axis_index("c")
        base = wid * b_per_w
        nchunks = b_per_w // CHUNK
        id_copies = [
            pltpu.async_copy(ids_hbm.at[pl.ds(base + j * CHUNK, CHUNK)],
                             idx_v.at[pl.ds(j * CHUNK, CHUNK)], sem0)
            for j in range(nchunks)]
        gathers = []
        for j in range(nchunks):
            id_copies[j].wait()
            gathers.append(pltpu.async_copy(
                token_hbm.at[idx_v.at[pl.ds(j * CHUNK, CHUNK)]],
                rows_v.at[pl.ds(j * CHUNK, CHUNK)], sem1))
        outs = []
        for j, g in enumerate(gathers):
            g.wait()
            outs.append(pltpu.async_copy(
                rows_v.at[pl.ds(j * CHUNK, CHUNK)],
                out_hbm.at[pl.ds(base + j * CHUNK, CHUNK)], sem0))
        for o in outs:
            o.wait()

    return sc_gather


def _tc_ln_body(x_ref, pos_ref, tt_ref, type_ref, g_ref, b_ref, o_ref):
    x = x_ref[...]
    t0 = type_ref[0:1, :]
    t1 = type_ref[1:2, :]
    n_rep = x.shape[0] // pos_ref.shape[0]
    pos = pos_ref[...]
    if n_rep > 1:
        pos = jnp.concatenate([pos] * n_rep, axis=0)
    e = x + pos + t0 + tt_ref[...].astype(jnp.float32) * (t1 - t0)
    mean = jnp.mean(e, axis=-1, keepdims=True)
    c = e - mean
    var = jnp.mean(c * c, axis=-1, keepdims=True)
    o_ref[...] = c * lax.rsqrt(var + 1e-12) * g_ref[...] + b_ref[...]


def _tc_ln(gathered, pos_table, tt, type_table, gamma, beta, seq_len):
    n = gathered.shape[0]
    r = ROWS_PER_STEP
    grid = n // r
    pos_rows = min(r, seq_len)
    pos_blocks = max(seq_len // r, 1)
    return pl.pallas_call(
        _tc_ln_body,
        grid=(grid,),
        in_specs=[
            pl.BlockSpec((r, HIDDEN), lambda g: (g, 0)),
            pl.BlockSpec((pos_rows, HIDDEN), lambda g: (g % pos_blocks, 0)),
            pl.BlockSpec((r, 1), lambda g: (g, 0)),
            pl.BlockSpec((2, HIDDEN), lambda g: (0, 0)),
            pl.BlockSpec((1, HIDDEN), lambda g: (0, 0)),
            pl.BlockSpec((1, HIDDEN), lambda g: (0, 0)),
        ],
        out_specs=pl.BlockSpec((r, HIDDEN), lambda g: (g, 0)),
        out_shape=jax.ShapeDtypeStruct((n, HIDDEN), jnp.float32),
    )(gathered, pos_table, tt, type_table, gamma, beta)


def kernel(input_ids, token_type_ids, token_table, pos_table, type_table,
           ln_gamma, ln_beta):
    b, s = input_ids.shape
    n = b * s
    ids = input_ids.reshape(n).astype(jnp.int32)
    tt = token_type_ids.reshape(n, 1).astype(jnp.int32)
    gathered = _make_sc_gather(n)(ids, token_table)
    out = _tc_ln(gathered, pos_table, tt, type_table,
                 ln_gamma.reshape(1, HIDDEN), ln_beta.reshape(1, HIDDEN), s)
    return out.reshape(b, s, HIDDEN)
